# baseline (speedup 1.0000x reference)
import functools

import jax
import jax.numpy as jnp
from jax import lax
from jax.experimental import pallas as pl
from jax.experimental.pallas import tpu as pltpu
from jax.experimental.pallas import tpu_sc as plsc

_ROW_BLOCK = 2048
_GATHER_CHUNK = 64


def _assign_kernel(res_ref, x2_ref, cb_ref, c2_ref, nn_ref, counts_ref):
    B = res_ref.shape[0]
    K = cb_ref.shape[0]

    @pl.when(pl.program_id(0) == 0)
    def _init():
        counts_ref[...] = jnp.zeros_like(counts_ref)

    residual = res_ref[...]
    x2 = x2_ref[...]
    c2 = c2_ref[...]
    dots = jax.lax.dot_general(
        residual.astype(jnp.bfloat16), cb_ref[...].astype(jnp.bfloat16),
        (((1,), (1,)), ((), ())),
        preferred_element_type=jnp.float32)
    dists = x2 - 2.0 * dots + c2
    m = jnp.min(dists, axis=1, keepdims=True)
    iota = jax.lax.broadcasted_iota(jnp.int32, (B, K), 1)
    nn = jnp.min(jnp.where(dists == m, iota, K), axis=1)
    nn_ref[...] = nn[:, None]
    onehot = (iota == nn[:, None]).astype(jnp.int32)
    counts_ref[...] += jnp.sum(onehot, axis=0)[None, :]


def _assign(residual, x2, cb, c2):
    n, d = residual.shape
    k = cb.shape[0]
    return pl.pallas_call(
        _assign_kernel,
        grid=(n // _ROW_BLOCK,),
        in_specs=[
            pl.BlockSpec((_ROW_BLOCK, d), lambda r: (r, 0)),
            pl.BlockSpec((_ROW_BLOCK, 1), lambda r: (r, 0)),
            pl.BlockSpec((k, d), lambda r: (0, 0)),
            pl.BlockSpec((1, k), lambda r: (0, 0)),
        ],
        out_specs=[
            pl.BlockSpec((_ROW_BLOCK, 1), lambda r: (r, 0)),
            pl.BlockSpec((1, k), lambda r: (0, 0)),
        ],
        out_shape=[
            jax.ShapeDtypeStruct((n, 1), jnp.int32),
            jax.ShapeDtypeStruct((1, k), jnp.int32),
        ],
    )(residual, x2, cb, c2)


@functools.lru_cache(maxsize=None)
def _make_sc_gather(n, k, d):
    info = plsc.get_sparse_core_info()
    nw = info.num_cores * info.num_subcores
    b_per_w = n // nw
    n_chunks = b_per_w // _GATHER_CHUNK
    mesh = plsc.VectorSubcoreMesh(core_axis_name="c", subcore_axis_name="s")

    @functools.partial(
        pl.kernel, mesh=mesh,
        out_type=jax.ShapeDtypeStruct((n, d), jnp.float32),
        scratch_types=[
            pltpu.VMEM((_GATHER_CHUNK,), jnp.int32),
            pltpu.VMEM((_GATHER_CHUNK, d), jnp.float32),
            pltpu.SemaphoreType.DMA,
        ],
    )
    def sc_gather(table_hbm, idx_hbm, out_hbm, idx_v, rows_v, sem):
        wid = lax.axis_index("s") * info.num_cores + lax.axis_index("c")
        base = wid * b_per_w
        for j in range(n_chunks):
            off = base + j * _GATHER_CHUNK
            pltpu.sync_copy(idx_hbm.at[pl.ds(off, _GATHER_CHUNK)], idx_v)
            pltpu.async_copy(table_hbm.at[idx_v], rows_v, sem).wait()
            pltpu.sync_copy(rows_v, out_hbm.at[pl.ds(off, _GATHER_CHUNK)])

    return sc_gather


def kernel(inputs, codebooks):
    batch, tokens, d = inputs.shape
    num_q, num_centroids, _ = codebooks.shape
    n = batch * tokens
    flat = jnp.reshape(inputs, (n, d))
    sc_gather = _make_sc_gather(n, num_centroids, d)

    residual = flat
    quantized = jnp.zeros_like(flat)
    loss = jnp.float32(0.0)
    denom = jnp.float32(n * d)
    x2 = jnp.sum(residual * residual, axis=1, keepdims=True)
    nn_list, counts_list = [], []
    for i in range(num_q):
        cb = codebooks[i]
        c2 = jnp.sum(cb * cb, axis=1)[None, :]
        nn, counts = _assign(residual, x2, cb, c2)
        e = sc_gather(cb, nn[:, 0])
        q = residual + (e - residual)
        quantized = quantized + q
        residual = residual - q
        x2 = jnp.sum(residual * residual, axis=1, keepdims=True)
        loss = loss + 1.25 * (jnp.sum(x2[:, 0]) / denom)
        nn_list.append(nn[:, 0])
        counts_list.append(counts[0])

    quantized = jnp.reshape(quantized, inputs.shape)
    qloss_arr = jnp.full(inputs.shape[:-1] + (1,), loss)
    nn_out = jnp.reshape(jnp.stack(nn_list, axis=0), (num_q, batch, tokens))
    cbs = jnp.reshape(codebooks, (num_q * num_centroids, d))
    counts_out = jnp.stack(counts_list, axis=0)
    return (quantized, qloss_arr, nn_out, cbs, counts_out)


# speedup vs baseline: 1.0096x; 1.0096x over previous
import functools

import jax
import jax.numpy as jnp
from jax import lax
from jax.experimental import pallas as pl
from jax.experimental.pallas import tpu as pltpu
from jax.experimental.pallas import tpu_sc as plsc

_ROW_BLOCK = 2048
_GATHER_CHUNK = 128


def _assign_kernel(res_ref, x2_ref, cb_ref, c2_ref, nn_ref, counts_ref):
    B = res_ref.shape[0]
    K = cb_ref.shape[0]

    @pl.when(pl.program_id(0) == 0)
    def _init():
        counts_ref[...] = jnp.zeros_like(counts_ref)

    residual = res_ref[...]
    x2 = x2_ref[...]
    c2 = c2_ref[...]
    dots = jax.lax.dot_general(
        residual.astype(jnp.bfloat16), cb_ref[...].astype(jnp.bfloat16),
        (((1,), (1,)), ((), ())),
        preferred_element_type=jnp.float32)
    dists = x2 - 2.0 * dots + c2
    m = jnp.min(dists, axis=1, keepdims=True)
    iota = jax.lax.broadcasted_iota(jnp.int32, (B, K), 1)
    nn = jnp.min(jnp.where(dists == m, iota, K), axis=1)
    nn_ref[...] = nn[:, None]
    onehot = (iota == nn[:, None]).astype(jnp.int32)
    counts_ref[...] += jnp.sum(onehot, axis=0)[None, :]


def _assign(residual, x2, cb, c2):
    n, d = residual.shape
    k = cb.shape[0]
    return pl.pallas_call(
        _assign_kernel,
        grid=(n // _ROW_BLOCK,),
        in_specs=[
            pl.BlockSpec((_ROW_BLOCK, d), lambda r: (r, 0)),
            pl.BlockSpec((_ROW_BLOCK, 1), lambda r: (r, 0)),
            pl.BlockSpec((k, d), lambda r: (0, 0)),
            pl.BlockSpec((1, k), lambda r: (0, 0)),
        ],
        out_specs=[
            pl.BlockSpec((_ROW_BLOCK, 1), lambda r: (r, 0)),
            pl.BlockSpec((1, k), lambda r: (0, 0)),
        ],
        out_shape=[
            jax.ShapeDtypeStruct((n, 1), jnp.int32),
            jax.ShapeDtypeStruct((1, k), jnp.int32),
        ],
    )(residual, x2, cb, c2)


@functools.lru_cache(maxsize=None)
def _make_sc_gather(n, k, d):
    info = plsc.get_sparse_core_info()
    nw = info.num_cores * info.num_subcores
    b_per_w = n // nw
    n_chunks = b_per_w // _GATHER_CHUNK
    mesh = plsc.VectorSubcoreMesh(core_axis_name="c", subcore_axis_name="s")

    @functools.partial(
        pl.kernel, mesh=mesh,
        out_type=jax.ShapeDtypeStruct((n, d), jnp.float32),
        scratch_types=[
            pltpu.VMEM((2, _GATHER_CHUNK), jnp.int32),
            pltpu.VMEM((2, _GATHER_CHUNK, d), jnp.float32),
            pltpu.SemaphoreType.DMA,
            pltpu.SemaphoreType.DMA,
            pltpu.SemaphoreType.DMA,
            pltpu.SemaphoreType.DMA,
        ],
    )
    def sc_gather(table_hbm, idx_hbm, out_hbm, idx_v, rows_v, g0, g1, o0, o1):
        wid = lax.axis_index("s") * info.num_cores + lax.axis_index("c")
        base = wid * b_per_w
        gsem = [g0, g1]
        osem = [o0, o1]
        gh = [None] * n_chunks
        oh = [None] * n_chunks
        for j in range(n_chunks):
            cur = j & 1
            if j >= 2:
                oh[j - 2].wait()
            off = base + j * _GATHER_CHUNK
            pltpu.sync_copy(idx_hbm.at[pl.ds(off, _GATHER_CHUNK)],
                            idx_v.at[cur])
            gh[j] = pltpu.async_copy(table_hbm.at[idx_v.at[cur]],
                                     rows_v.at[cur], gsem[cur])
            if j >= 1:
                prev = (j - 1) & 1
                gh[j - 1].wait()
                off_p = base + (j - 1) * _GATHER_CHUNK
                oh[j - 1] = pltpu.async_copy(
                    rows_v.at[prev],
                    out_hbm.at[pl.ds(off_p, _GATHER_CHUNK)], osem[prev])
        last = n_chunks - 1
        gh[last].wait()
        oh[last] = pltpu.async_copy(
            rows_v.at[last & 1],
            out_hbm.at[pl.ds(base + last * _GATHER_CHUNK, _GATHER_CHUNK)],
            osem[last & 1])
        oh[last - 1].wait()
        oh[last].wait()

    return sc_gather


def kernel(inputs, codebooks):
    batch, tokens, d = inputs.shape
    num_q, num_centroids, _ = codebooks.shape
    n = batch * tokens
    flat = jnp.reshape(inputs, (n, d))
    sc_gather = _make_sc_gather(n, num_centroids, d)

    residual = flat
    quantized = jnp.zeros_like(flat)
    loss = jnp.float32(0.0)
    denom = jnp.float32(n * d)
    x2 = jnp.sum(residual * residual, axis=1, keepdims=True)
    nn_list, counts_list = [], []
    for i in range(num_q):
        cb = codebooks[i]
        c2 = jnp.sum(cb * cb, axis=1)[None, :]
        nn, counts = _assign(residual, x2, cb, c2)
        e = sc_gather(cb, nn[:, 0])
        q = residual + (e - residual)
        quantized = quantized + q
        residual = residual - q
        x2 = jnp.sum(residual * residual, axis=1, keepdims=True)
        loss = loss + 1.25 * (jnp.sum(x2[:, 0]) / denom)
        nn_list.append(nn[:, 0])
        counts_list.append(counts[0])

    quantized = jnp.reshape(quantized, inputs.shape)
    qloss_arr = jnp.full(inputs.shape[:-1] + (1,), loss)
    nn_out = jnp.reshape(jnp.stack(nn_list, axis=0), (num_q, batch, tokens))
    cbs = jnp.reshape(codebooks, (num_q * num_centroids, d))
    counts_out = jnp.stack(counts_list, axis=0)
    return (quantized, qloss_arr, nn_out, cbs, counts_out)


# speedup vs baseline: 1.0967x; 1.0863x over previous
import jax
import jax.numpy as jnp
from jax.experimental import pallas as pl

_ROW_BLOCK = 1024


def _stage_kernel(res_ref, x2_ref, cb_ref, cbp_ref, c2_ref,
                  q_ref, resout_ref, nn_ref, counts_ref, sse_ref):
    B = res_ref.shape[0]
    K = cbp_ref.shape[1]

    @pl.when(pl.program_id(0) == 0)
    def _init():
        counts_ref[...] = jnp.zeros_like(counts_ref)
        sse_ref[...] = jnp.zeros_like(sse_ref)

    residual = res_ref[...]
    x2 = x2_ref[...]
    c2 = c2_ref[...]
    dots = jax.lax.dot_general(
        residual.astype(jnp.bfloat16), cb_ref[...].astype(jnp.bfloat16),
        (((1,), (1,)), ((), ())),
        preferred_element_type=jnp.float32)
    dists = x2 - 2.0 * dots + c2
    m = jnp.min(dists, axis=1, keepdims=True)
    iota = jax.lax.broadcasted_iota(jnp.int32, (B, K), 1)
    nn = jnp.min(jnp.where(dists == m, iota, K), axis=1)
    nn_ref[...] = nn[:, None]
    onehot = (iota == nn[:, None]).astype(jnp.float32)
    counts_ref[...] += jnp.sum(onehot, axis=0).astype(jnp.int32)[None, :]
    ohb = onehot.astype(jnp.bfloat16)
    e_hi = jnp.dot(ohb, cbp_ref[0], preferred_element_type=jnp.float32)
    e_mid = jnp.dot(ohb, cbp_ref[1], preferred_element_type=jnp.float32)
    e_lo = jnp.dot(ohb, cbp_ref[2], preferred_element_type=jnp.float32)
    e = (e_hi + e_mid) + e_lo
    diff = residual - e
    sse_ref[...] += jnp.sum(diff * diff)
    q = residual + (e - residual)
    q_ref[...] = q
    resout_ref[...] = residual - q


def _stage(residual, x2, cb, cb_parts, c2):
    n, d = residual.shape
    k = cb_parts.shape[1]
    return pl.pallas_call(
        _stage_kernel,
        grid=(n // _ROW_BLOCK,),
        in_specs=[
            pl.BlockSpec((_ROW_BLOCK, d), lambda r: (r, 0)),
            pl.BlockSpec((_ROW_BLOCK, 1), lambda r: (r, 0)),
            pl.BlockSpec((k, d), lambda r: (0, 0)),
            pl.BlockSpec((3, k, d), lambda r: (0, 0, 0)),
            pl.BlockSpec((1, k), lambda r: (0, 0)),
        ],
        out_specs=[
            pl.BlockSpec((_ROW_BLOCK, d), lambda r: (r, 0)),
            pl.BlockSpec((_ROW_BLOCK, d), lambda r: (r, 0)),
            pl.BlockSpec((_ROW_BLOCK, 1), lambda r: (r, 0)),
            pl.BlockSpec((1, k), lambda r: (0, 0)),
            pl.BlockSpec((1, 1), lambda r: (0, 0)),
        ],
        out_shape=[
            jax.ShapeDtypeStruct((n, d), jnp.float32),
            jax.ShapeDtypeStruct((n, d), jnp.float32),
            jax.ShapeDtypeStruct((n, 1), jnp.int32),
            jax.ShapeDtypeStruct((1, k), jnp.int32),
            jax.ShapeDtypeStruct((1, 1), jnp.float32),
        ],
    )(residual, x2, cb, cb_parts, c2)


def kernel(inputs, codebooks):
    batch, tokens, d = inputs.shape
    num_q, num_centroids, _ = codebooks.shape
    n = batch * tokens
    flat = jnp.reshape(inputs, (n, d))

    mask = jnp.uint32(0xFFFF0000)
    u = jax.lax.bitcast_convert_type(codebooks, jnp.uint32)
    hi_f = jax.lax.bitcast_convert_type(u & mask, jnp.float32)
    r1 = codebooks - hi_f
    u1 = jax.lax.bitcast_convert_type(r1, jnp.uint32)
    mid_f = jax.lax.bitcast_convert_type(u1 & mask, jnp.float32)
    lo_f = r1 - mid_f
    parts = jnp.stack([hi_f.astype(jnp.bfloat16), mid_f.astype(jnp.bfloat16),
                       lo_f.astype(jnp.bfloat16)], axis=1)

    residual = flat
    quantized = jnp.zeros_like(flat)
    loss = jnp.float32(0.0)
    denom = jnp.float32(n * d)
    nn_list, counts_list = [], []
    for i in range(num_q):
        cb = codebooks[i]
        c2 = jnp.sum(cb * cb, axis=1)[None, :]
        x2 = jnp.sum(residual * residual, axis=1, keepdims=True)
        q, residual, nn, counts, sse = _stage(residual, x2, cb, parts[i], c2)
        quantized = quantized + q
        loss = loss + 1.25 * (sse[0, 0] / denom)
        nn_list.append(nn[:, 0])
        counts_list.append(counts[0])

    quantized = jnp.reshape(quantized, inputs.shape)
    qloss_arr = jnp.full(inputs.shape[:-1] + (1,), loss)
    nn_out = jnp.reshape(jnp.stack(nn_list, axis=0), (num_q, batch, tokens))
    cbs = jnp.reshape(codebooks, (num_q * num_centroids, d))
    counts_out = jnp.stack(counts_list, axis=0)
    return (quantized, qloss_arr, nn_out, cbs, counts_out)
